# Initial kernel scaffold; baseline (speedup 1.0000x reference)
import jax
import jax.numpy as jnp
from jax.experimental import pallas as pl


def kernel(x_t_feats, x_t_coords, tex_feats, tex_coords, shape_feats, shape_coords, t, cond, point_feats, point_coords, point_labels, coords_len_list, seg_weight, W_in, W_shape, W_cond, W_out):
    raise NotImplementedError("write your pallas kernel here")


# speedup vs baseline: 2.9106x; 2.9106x over previous
import jax
import jax.numpy as jnp
from jax.experimental import pallas as pl


def _mlp_kernel(x_ref, s_ref, c_ref, t_ref, cond_ref, lab_ref, segw_ref,
                win_ref, wsh_ref, wcond_ref, wout_ref, of_ref, oc_ref):
    h = jnp.dot(x_ref[...], win_ref[...], preferred_element_type=jnp.float32)
    h = h + jnp.dot(s_ref[...], wsh_ref[...], preferred_element_type=jnp.float32)
    bias = jnp.dot(cond_ref[0], wcond_ref[...],
                   preferred_element_type=jnp.float32)
    n_pos = jnp.sum((lab_ref[...] == 1).astype(jnp.float32))
    pe_mean = segw_ref[...] * (n_pos * 0.1)
    h = (h + bias + pe_mean) * (1.0 + t_ref[0, 0, 0])
    h = jax.nn.gelu(h)
    of_ref[...] = jnp.dot(h, wout_ref[...], preferred_element_type=jnp.float32)
    oc_ref[...] = c_ref[...]


def kernel(x_t_feats, x_t_coords, tex_feats, tex_coords, shape_feats,
           shape_coords, t, cond, point_feats, point_coords, point_labels,
           coords_len_list, seg_weight, W_in, W_shape, W_cond, W_out):
    nseg = coords_len_list.shape[0]
    ntok, d = x_t_feats.shape
    L = ntok // nseg
    dm = W_in.shape[1]
    dc = cond.shape[1]

    t2 = t.reshape(nseg, 1, 1).astype(jnp.float32)
    cond3 = cond.reshape(nseg, 1, dc)
    labels = point_labels.reshape(1, -1).astype(jnp.int32)

    outs_f, outs_c = pl.pallas_call(
        _mlp_kernel,
        grid=(nseg,),
        in_specs=[
            pl.BlockSpec((L, d), lambda i: (i, 0)),
            pl.BlockSpec((L, d), lambda i: (i, 0)),
            pl.BlockSpec((L, 4), lambda i: (i, 0)),
            pl.BlockSpec((1, 1, 1), lambda i: (i, 0, 0)),
            pl.BlockSpec((1, 1, dc), lambda i: (i, 0, 0)),
            pl.BlockSpec(labels.shape, lambda i: (0, 0)),
            pl.BlockSpec((1, dm), lambda i: (0, 0)),
            pl.BlockSpec((d, dm), lambda i: (0, 0)),
            pl.BlockSpec((d, dm), lambda i: (0, 0)),
            pl.BlockSpec((dc, dm), lambda i: (0, 0)),
            pl.BlockSpec((dm, d), lambda i: (0, 0)),
        ],
        out_specs=[
            pl.BlockSpec((L, d), lambda i: (i, 0)),
            pl.BlockSpec((L, 4), lambda i: (i, 0)),
        ],
        out_shape=[
            jax.ShapeDtypeStruct((ntok, d), jnp.float32),
            jax.ShapeDtypeStruct((ntok, 4), x_t_coords.dtype),
        ],
    )(x_t_feats, shape_feats, x_t_coords, t2, cond3, labels, seg_weight,
      W_in, W_shape, W_cond, W_out)
    return outs_f, outs_c


# speedup vs baseline: 3.6878x; 1.2670x over previous
import jax
import jax.numpy as jnp
from jax.experimental import pallas as pl

_C1 = 0.7978845608028654
_C3 = 0.044715


def _mlp_kernel(xs_ref, c_ref, t_ref, cond_ref, lab_ref, segw_ref,
                wis_ref, wcond_ref, wout_ref, of_ref, oc_ref):
    h = jnp.dot(xs_ref[...], wis_ref[...], preferred_element_type=jnp.float32)
    bias = jnp.dot(cond_ref[0], wcond_ref[...],
                   preferred_element_type=jnp.float32)
    n_pos = jnp.sum((lab_ref[...] == 1).astype(jnp.float32))
    scale = 1.0 + t_ref[0, 0, 0]
    b2s = (bias + segw_ref[...] * (n_pos * 0.1)) * scale
    g = h * scale + b2s
    u = jnp.tanh((g * _C1) * (1.0 + _C3 * (g * g)))
    a = g * u + g
    of_ref[...] = jnp.dot(a, wout_ref[...], preferred_element_type=jnp.float32)
    oc_ref[...] = c_ref[...]


def kernel(x_t_feats, x_t_coords, tex_feats, tex_coords, shape_feats,
           shape_coords, t, cond, point_feats, point_coords, point_labels,
           coords_len_list, seg_weight, W_in, W_shape, W_cond, W_out):
    nseg = coords_len_list.shape[0]
    ntok, d = x_t_feats.shape
    L = ntok // nseg
    dm = W_in.shape[1]
    dc = cond.shape[1]

    xs = jnp.concatenate([x_t_feats, shape_feats], axis=1)
    W_is = jnp.concatenate([W_in, W_shape], axis=0)
    W_out_half = W_out * 0.5
    t2 = t.reshape(nseg, 1, 1).astype(jnp.float32)
    cond3 = cond.reshape(nseg, 1, dc)
    labels = point_labels.reshape(1, -1).astype(jnp.int32)

    outs_f, outs_c = pl.pallas_call(
        _mlp_kernel,
        grid=(nseg,),
        in_specs=[
            pl.BlockSpec((L, 2 * d), lambda i: (i, 0)),
            pl.BlockSpec((L, 4), lambda i: (i, 0)),
            pl.BlockSpec((1, 1, 1), lambda i: (i, 0, 0)),
            pl.BlockSpec((1, 1, dc), lambda i: (i, 0, 0)),
            pl.BlockSpec(labels.shape, lambda i: (0, 0)),
            pl.BlockSpec((1, dm), lambda i: (0, 0)),
            pl.BlockSpec((2 * d, dm), lambda i: (0, 0)),
            pl.BlockSpec((dc, dm), lambda i: (0, 0)),
            pl.BlockSpec((dm, d), lambda i: (0, 0)),
        ],
        out_specs=[
            pl.BlockSpec((L, d), lambda i: (i, 0)),
            pl.BlockSpec((L, 4), lambda i: (i, 0)),
        ],
        out_shape=[
            jax.ShapeDtypeStruct((ntok, d), jnp.float32),
            jax.ShapeDtypeStruct((ntok, 4), x_t_coords.dtype),
        ],
    )(xs, x_t_coords, t2, cond3, labels, seg_weight,
      W_is, W_cond, W_out_half)
    return outs_f, outs_c


# speedup vs baseline: 4.7366x; 1.2844x over previous
import jax
import jax.numpy as jnp
from jax.experimental import pallas as pl

_C1 = 0.7978845608028654
_C3 = 0.044715
_C13 = _C1 * _C3


def _mlp_kernel(xs_ref, c_ref, t_ref, cond_ref, lab_ref, segw_ref,
                wis_ref, wcond_ref, wout_ref, of_ref, oc_ref):
    scale = 1.0 + t_ref[0, 0, 0]
    wis_s = (wis_ref[...] * scale).astype(jnp.bfloat16)
    h = jnp.dot(xs_ref[...], wis_s, preferred_element_type=jnp.float32)
    bias = jnp.dot(cond_ref[0], wcond_ref[...],
                   preferred_element_type=jnp.float32)
    n_pos = jnp.sum((lab_ref[...] == 1).astype(jnp.float32))
    b2s = (bias + segw_ref[...] * (n_pos * 0.1)) * scale
    g = h + b2s
    gg = g * g
    u = jnp.tanh(g * (_C1 + _C13 * gg))
    a = g * u + g
    of_ref[...] = jnp.dot(a, wout_ref[...], preferred_element_type=jnp.float32)
    oc_ref[...] = c_ref[...]


def kernel(x_t_feats, x_t_coords, tex_feats, tex_coords, shape_feats,
           shape_coords, t, cond, point_feats, point_coords, point_labels,
           coords_len_list, seg_weight, W_in, W_shape, W_cond, W_out):
    nseg = coords_len_list.shape[0]
    ntok, d = x_t_feats.shape
    L = ntok // nseg
    dm = W_in.shape[1]
    dc = cond.shape[1]

    xs = jnp.concatenate([x_t_feats, shape_feats],
                         axis=1).astype(jnp.bfloat16)
    W_is = jnp.concatenate([W_in, W_shape], axis=0)
    W_out_half = W_out * 0.5
    t2 = t.reshape(nseg, 1, 1).astype(jnp.float32)
    cond3 = cond.reshape(nseg, 1, dc)
    labels = point_labels.reshape(1, -1).astype(jnp.int32)

    outs_f, outs_c = pl.pallas_call(
        _mlp_kernel,
        grid=(nseg,),
        in_specs=[
            pl.BlockSpec((L, 2 * d), lambda i: (i, 0)),
            pl.BlockSpec((L, 4), lambda i: (i, 0)),
            pl.BlockSpec((1, 1, 1), lambda i: (i, 0, 0)),
            pl.BlockSpec((1, 1, dc), lambda i: (i, 0, 0)),
            pl.BlockSpec(labels.shape, lambda i: (0, 0)),
            pl.BlockSpec((1, dm), lambda i: (0, 0)),
            pl.BlockSpec((2 * d, dm), lambda i: (0, 0)),
            pl.BlockSpec((dc, dm), lambda i: (0, 0)),
            pl.BlockSpec((dm, d), lambda i: (0, 0)),
        ],
        out_specs=[
            pl.BlockSpec((L, d), lambda i: (i, 0)),
            pl.BlockSpec((L, 4), lambda i: (i, 0)),
        ],
        out_shape=[
            jax.ShapeDtypeStruct((ntok, d), jnp.float32),
            jax.ShapeDtypeStruct((ntok, 4), x_t_coords.dtype),
        ],
    )(xs, x_t_coords, t2, cond3, labels, seg_weight,
      W_is, W_cond, W_out_half)
    return outs_f, outs_c


# speedup vs baseline: 5.2078x; 1.0995x over previous
import jax
import jax.numpy as jnp
from jax.experimental import pallas as pl
from jax.experimental.pallas import tpu as pltpu

_C1 = 0.7978845608028654
_C3 = 0.044715
_C13 = _C1 * _C3


def _mlp_kernel(x_ref, s_ref, c_ref, t_ref, cond_ref, lab_ref, segw_ref,
                win_ref, wsh_ref, wcond_ref, wout_ref, of_ref, oc_ref,
                bias_scr):
    i = pl.program_id(0)

    @pl.when(i == 0)
    def _():
        bias_scr[...] = jnp.dot(cond_ref[...], wcond_ref[...],
                                preferred_element_type=jnp.float32)

    scale = 1.0 + t_ref[0, 0, 0]
    wis = jnp.concatenate([win_ref[...], wsh_ref[...]], axis=0)
    wis_s = (wis * scale).astype(jnp.bfloat16)
    xs = jnp.concatenate([x_ref[...].astype(jnp.bfloat16),
                          s_ref[...].astype(jnp.bfloat16)], axis=1)
    h = jnp.dot(xs, wis_s, preferred_element_type=jnp.float32)

    n_pos = jnp.sum((lab_ref[...] == 1).astype(jnp.float32))
    b2s = (bias_scr[pl.ds(i, 1), :]
           + segw_ref[...] * (n_pos * 0.1)) * scale
    g = (h + b2s).astype(jnp.bfloat16)
    gg = g * g
    u = jnp.tanh(g * (jnp.bfloat16(_C1) + jnp.bfloat16(_C13) * gg))
    a = g * u + g
    wo = (wout_ref[...] * 0.5).astype(jnp.bfloat16)
    of_ref[...] = jnp.dot(a, wo, preferred_element_type=jnp.float32)
    oc_ref[...] = c_ref[...]


def kernel(x_t_feats, x_t_coords, tex_feats, tex_coords, shape_feats,
           shape_coords, t, cond, point_feats, point_coords, point_labels,
           coords_len_list, seg_weight, W_in, W_shape, W_cond, W_out):
    nseg = coords_len_list.shape[0]
    ntok, d = x_t_feats.shape
    L = ntok // nseg
    dm = W_in.shape[1]
    dc = cond.shape[1]

    t2 = t.reshape(nseg, 1, 1).astype(jnp.float32)
    labels = point_labels.reshape(1, -1).astype(jnp.int32)

    outs_f, outs_c = pl.pallas_call(
        _mlp_kernel,
        grid=(nseg,),
        in_specs=[
            pl.BlockSpec((L, d), lambda i: (i, 0)),
            pl.BlockSpec((L, d), lambda i: (i, 0)),
            pl.BlockSpec((L, 4), lambda i: (i, 0)),
            pl.BlockSpec((1, 1, 1), lambda i: (i, 0, 0)),
            pl.BlockSpec((nseg, dc), lambda i: (0, 0)),
            pl.BlockSpec(labels.shape, lambda i: (0, 0)),
            pl.BlockSpec((1, dm), lambda i: (0, 0)),
            pl.BlockSpec((d, dm), lambda i: (0, 0)),
            pl.BlockSpec((d, dm), lambda i: (0, 0)),
            pl.BlockSpec((dc, dm), lambda i: (0, 0)),
            pl.BlockSpec((dm, d), lambda i: (0, 0)),
        ],
        out_specs=[
            pl.BlockSpec((L, d), lambda i: (i, 0)),
            pl.BlockSpec((L, 4), lambda i: (i, 0)),
        ],
        out_shape=[
            jax.ShapeDtypeStruct((ntok, d), jnp.float32),
            jax.ShapeDtypeStruct((ntok, 4), x_t_coords.dtype),
        ],
        scratch_shapes=[pltpu.VMEM((nseg, dm), jnp.float32)],
    )(x_t_feats, shape_feats, x_t_coords, t2, cond, labels, seg_weight,
      W_in, W_shape, W_cond, W_out)
    return outs_f, outs_c


# speedup vs baseline: 5.3419x; 1.0257x over previous
import jax
import jax.numpy as jnp
from jax.experimental import pallas as pl
from jax.experimental.pallas import tpu as pltpu

_C1 = 0.7978845608028654
_C3 = 0.044715
_C13 = _C1 * _C3

SEGS_PER_STEP = 2


def _mlp_kernel(x_ref, s_ref, c_ref, t_ref, cond_ref, lab_ref, segw_ref,
                win_ref, wsh_ref, wcond_ref, wout_ref, of_ref, oc_ref,
                bias_scr):
    i = pl.program_id(0)
    L = x_ref.shape[0] // SEGS_PER_STEP

    @pl.when(i == 0)
    def _():
        bias_scr[...] = jnp.dot(cond_ref[...], wcond_ref[...],
                                preferred_element_type=jnp.float32)

    n_pos = jnp.sum((lab_ref[...] == 1).astype(jnp.float32))
    pe = segw_ref[...] * (n_pos * 0.1)
    wis = jnp.concatenate([win_ref[...], wsh_ref[...]], axis=0)
    wo = (wout_ref[...] * 0.5).astype(jnp.bfloat16)

    for j in range(SEGS_PER_STEP):
        scale = 1.0 + t_ref[j, 0, 0]
        wis_s = (wis * scale).astype(jnp.bfloat16)
        rows = pl.ds(j * L, L)
        xs = jnp.concatenate([x_ref[rows, :].astype(jnp.bfloat16),
                              s_ref[rows, :].astype(jnp.bfloat16)], axis=1)
        h = jnp.dot(xs, wis_s, preferred_element_type=jnp.float32)
        b2s = (bias_scr[pl.ds(i * SEGS_PER_STEP + j, 1), :] + pe) * scale
        g = (h + b2s).astype(jnp.bfloat16)
        gg = g * g
        u = jnp.tanh(g * (jnp.bfloat16(_C1) + jnp.bfloat16(_C13) * gg))
        a = g * u + g
        of_ref[rows, :] = jnp.dot(a, wo, preferred_element_type=jnp.float32)
    oc_ref[...] = c_ref[...]


def kernel(x_t_feats, x_t_coords, tex_feats, tex_coords, shape_feats,
           shape_coords, t, cond, point_feats, point_coords, point_labels,
           coords_len_list, seg_weight, W_in, W_shape, W_cond, W_out):
    nseg = coords_len_list.shape[0]
    ntok, d = x_t_feats.shape
    dm = W_in.shape[1]
    dc = cond.shape[1]
    nstep = nseg // SEGS_PER_STEP
    rows_per_step = ntok // nstep

    t2 = t.reshape(nseg, 1, 1).astype(jnp.float32)
    labels = point_labels.reshape(1, -1).astype(jnp.int32)

    outs_f, outs_c = pl.pallas_call(
        _mlp_kernel,
        grid=(nstep,),
        in_specs=[
            pl.BlockSpec((rows_per_step, d), lambda i: (i, 0)),
            pl.BlockSpec((rows_per_step, d), lambda i: (i, 0)),
            pl.BlockSpec((rows_per_step, 4), lambda i: (i, 0)),
            pl.BlockSpec((SEGS_PER_STEP, 1, 1), lambda i: (i, 0, 0)),
            pl.BlockSpec((nseg, dc), lambda i: (0, 0)),
            pl.BlockSpec(labels.shape, lambda i: (0, 0)),
            pl.BlockSpec((1, dm), lambda i: (0, 0)),
            pl.BlockSpec((d, dm), lambda i: (0, 0)),
            pl.BlockSpec((d, dm), lambda i: (0, 0)),
            pl.BlockSpec((dc, dm), lambda i: (0, 0)),
            pl.BlockSpec((dm, d), lambda i: (0, 0)),
        ],
        out_specs=[
            pl.BlockSpec((rows_per_step, d), lambda i: (i, 0)),
            pl.BlockSpec((rows_per_step, 4), lambda i: (i, 0)),
        ],
        out_shape=[
            jax.ShapeDtypeStruct((ntok, d), jnp.float32),
            jax.ShapeDtypeStruct((ntok, 4), x_t_coords.dtype),
        ],
        scratch_shapes=[pltpu.VMEM((nseg, dm), jnp.float32)],
    )(x_t_feats, shape_feats, x_t_coords, t2, cond, labels, seg_weight,
      W_in, W_shape, W_cond, W_out)
    return outs_f, outs_c
